# baseline (speedup 1.0000x reference)
from functools import partial

import jax
import jax.numpy as jnp
from jax.experimental import pallas as pl
from jax.experimental.pallas import tpu as pltpu

_C = 3
_K = 3
_EPS = 1e-5
_VMEM_BYTES = 64 * 1024 * 1024
_DS = (-2, -1, 0, 1, 2)


def _conv_stats_kernel(wv_ref, x_ref, y_ref, stats_ref, sacc_ref, qacc_ref):
    step = pl.program_id(1)
    inner = pl.num_programs(1)
    rows, length = x_ref.shape

    x = x_ref[...]
    col = jax.lax.broadcasted_iota(jnp.int32, (rows, length), 1)
    xm = jnp.where(col == 0, 0.0, pltpu.roll(x, shift=1, axis=1))
    xp = jnp.where(col == length - 1, 0.0,
                   pltpu.roll(x, shift=length - 1, axis=1))
    taps = (xm, x, xp)

    y = None
    for di, d in enumerate(_DS):
        p = None
        for k in range(_K):
            t = wv_ref[:, 3 * di + k: 3 * di + k + 1] * taps[k]
            p = t if p is None else p + t
        if d != 0:
            p = pltpu.roll(p, shift=d % rows, axis=0)
        y = p if y is None else y + p

    @pl.when(step == 0)
    def _init():
        sacc_ref[...] = jnp.zeros_like(sacc_ref)
        qacc_ref[...] = jnp.zeros_like(qacc_ref)

    y_ref[...] = y.astype(y_ref.dtype)
    sacc_ref[...] = sacc_ref[...] + y
    qacc_ref[...] = qacc_ref[...] + y * y

    @pl.when(step == inner - 1)
    def _flush():
        rowc = jax.lax.broadcasted_iota(jnp.int32, (rows, length), 0) % _C
        s = sacc_ref[...]
        q = qacc_ref[...]
        for co in range(_C):
            m = rowc == co
            stats_ref[0, 0, co] = jnp.sum(jnp.where(m, s, 0.0), axis=0)
            stats_ref[0, 1, co] = jnp.sum(jnp.where(m, q, 0.0), axis=0)


def _affine_kernel(af_ref, y_ref, o_ref):
    sc = af_ref[:, 0:1]
    sh = af_ref[:, 1:2]
    o_ref[...] = y_ref[...].astype(jnp.float32) * sc + sh


def _largest_tile(n, cap):
    best = 1
    for t in range(1, min(n, cap) + 1):
        if n % t == 0:
            best = t
    return best


@jax.jit
def _forward(x, w, gamma, beta):
    n, c_in, length = x.shape
    assert c_in == _C and w.shape == (_C, _C, _K)

    tile_n = _largest_tile(n, 256)
    n_tiles = n // tile_n
    n_split = 2 if n_tiles % 2 == 0 else 1
    inner = n_tiles // n_split
    rows = _C * tile_n

    w32 = w.astype(jnp.float32)
    x2 = x.reshape(n * _C, length)

    rmod = jnp.arange(rows) % _C
    cols = []
    for d in _DS:
        dest = rmod + d
        valid = (dest >= 0) & (dest < _C)
        destc = jnp.clip(dest, 0, _C - 1)
        for k in range(_K):
            cols.append(jnp.where(valid, w32[destc, rmod, k], 0.0))
    cols.append(jnp.zeros((rows,), jnp.float32))
    wv = jnp.stack(cols, axis=1)

    y, stats = pl.pallas_call(
        _conv_stats_kernel,
        out_shape=(
            jax.ShapeDtypeStruct((n * _C, length), jnp.bfloat16),
            jax.ShapeDtypeStruct((n_split, 2, _C, length), jnp.float32),
        ),
        grid=(n_split, inner),
        in_specs=[
            pl.BlockSpec((rows, 16), lambda c, i: (0, 0)),
            pl.BlockSpec((rows, length), lambda c, i: (c * inner + i, 0)),
        ],
        out_specs=(
            pl.BlockSpec((rows, length), lambda c, i: (c * inner + i, 0)),
            pl.BlockSpec((1, 2, _C, length), lambda c, i: (c, 0, 0, 0)),
        ),
        scratch_shapes=[
            pltpu.VMEM((rows, length), jnp.float32),
            pltpu.VMEM((rows, length), jnp.float32),
        ],
        compiler_params=pltpu.CompilerParams(
            dimension_semantics=("parallel", "arbitrary"),
            vmem_limit_bytes=_VMEM_BYTES),
    )(wv, x2)

    count = jnp.float32(n * length)
    ch_sum = jnp.sum(stats[:, 0], axis=(0, 2))
    ch_sumsq = jnp.sum(stats[:, 1], axis=(0, 2))
    mean = ch_sum / count
    var = jnp.maximum(ch_sumsq / count - mean * mean, 0.0)
    inv = jax.lax.rsqrt(var + _EPS)
    scale = gamma.astype(jnp.float32) * inv
    shift = beta.astype(jnp.float32) - mean * scale

    tile2 = _largest_tile(n, 256)
    rows2 = _C * tile2
    rmod2 = jnp.arange(rows2) % _C
    af = jnp.stack([scale[rmod2], shift[rmod2]], axis=1)

    out2 = pl.pallas_call(
        _affine_kernel,
        out_shape=jax.ShapeDtypeStruct((n * _C, length), x.dtype),
        grid=(n // tile2,),
        in_specs=[
            pl.BlockSpec((rows2, 2), lambda i: (0, 0)),
            pl.BlockSpec((rows2, length), lambda i: (i, 0)),
        ],
        out_specs=pl.BlockSpec((rows2, length), lambda i: (i, 0)),
        compiler_params=pltpu.CompilerParams(
            dimension_semantics=("parallel",),
            vmem_limit_bytes=_VMEM_BYTES),
    )(af, y)
    return out2.reshape(n, _C, length)


def kernel(x, w, gamma, beta):
    return _forward(x, w, gamma, beta)


# speedup vs baseline: 1.7446x; 1.7446x over previous
from functools import partial

import jax
import jax.numpy as jnp
from jax.experimental import pallas as pl
from jax.experimental.pallas import tpu as pltpu

_C = 3
_K = 3
_EPS = 1e-5
_VMEM_BYTES = 64 * 1024 * 1024


def _conv_stats_kernel(w_ref, x_ref, y_ref, stats_ref, sacc_ref, qacc_ref):
    step = pl.program_id(1)
    inner = pl.num_programs(1)
    tile_n, cl = x_ref.shape
    length = cl // _C

    col = jax.lax.broadcasted_iota(jnp.int32, (tile_n, length), 1)
    first = col == 0
    last = col == length - 1

    accs = [None, None, None]
    for ci in range(_C):
        xc = x_ref[:, ci * length:(ci + 1) * length]
        xm = jnp.where(first, 0.0, pltpu.roll(xc, shift=1, axis=1))
        xp = jnp.where(last, 0.0, pltpu.roll(xc, shift=length - 1, axis=1))
        for co in range(_C):
            base = (co * _C + ci) * _K
            t = w_ref[base] * xm + w_ref[base + 1] * xc + w_ref[base + 2] * xp
            accs[co] = t if ci == 0 else accs[co] + t

    @pl.when(step == 0)
    def _init():
        sacc_ref[...] = jnp.zeros_like(sacc_ref)
        qacc_ref[...] = jnp.zeros_like(qacc_ref)

    for co in range(_C):
        y = accs[co]
        y_ref[:, co * length:(co + 1) * length] = y.astype(y_ref.dtype)
        sacc_ref[co] = sacc_ref[co] + y
        qacc_ref[co] = qacc_ref[co] + y * y

    @pl.when(step == inner - 1)
    def _flush():
        for co in range(_C):
            stats_ref[0, 0, co] = jnp.sum(sacc_ref[co], axis=0)
            stats_ref[0, 1, co] = jnp.sum(qacc_ref[co], axis=0)


def _affine_kernel(sc_ref, sh_ref, y_ref, o_ref):
    tile2, cl = y_ref.shape
    length = cl // _C
    for co in range(_C):
        sl = slice(co * length, (co + 1) * length)
        o_ref[:, sl] = (y_ref[:, sl].astype(jnp.float32) * sc_ref[co]
                        + sh_ref[co])


def _largest_tile(n, cap):
    best = 1
    for t in range(1, min(n, cap) + 1):
        if n % t == 0:
            best = t
    return best


@jax.jit
def _forward(x, w, gamma, beta):
    n, c_in, length = x.shape
    assert c_in == _C and w.shape == (_C, _C, _K)

    tile_n = _largest_tile(n, 256)
    n_tiles = n // tile_n
    n_split = 2 if n_tiles % 2 == 0 else 1
    inner = n_tiles // n_split

    w_flat = w.astype(jnp.float32).reshape(-1)
    x2 = x.reshape(n, _C * length)
    smem = pl.BlockSpec(memory_space=pltpu.MemorySpace.SMEM)

    y, stats = pl.pallas_call(
        _conv_stats_kernel,
        out_shape=(
            jax.ShapeDtypeStruct((n, _C * length), jnp.bfloat16),
            jax.ShapeDtypeStruct((n_split, 2, _C, length), jnp.float32),
        ),
        grid=(n_split, inner),
        in_specs=[
            smem,
            pl.BlockSpec((tile_n, _C * length),
                         lambda c, i: (c * inner + i, 0)),
        ],
        out_specs=(
            pl.BlockSpec((tile_n, _C * length),
                         lambda c, i: (c * inner + i, 0)),
            pl.BlockSpec((1, 2, _C, length), lambda c, i: (c, 0, 0, 0)),
        ),
        scratch_shapes=[
            pltpu.VMEM((_C, tile_n, length), jnp.float32),
            pltpu.VMEM((_C, tile_n, length), jnp.float32),
        ],
        compiler_params=pltpu.CompilerParams(
            dimension_semantics=("parallel", "arbitrary"),
            vmem_limit_bytes=_VMEM_BYTES),
    )(w_flat, x2)

    count = jnp.float32(n * length)
    ch_sum = jnp.sum(stats[:, 0], axis=(0, 2))
    ch_sumsq = jnp.sum(stats[:, 1], axis=(0, 2))
    mean = ch_sum / count
    var = jnp.maximum(ch_sumsq / count - mean * mean, 0.0)
    inv = jax.lax.rsqrt(var + _EPS)
    scale = gamma.astype(jnp.float32) * inv
    shift = beta.astype(jnp.float32) - mean * scale

    tile2 = _largest_tile(n, 512)
    out2 = pl.pallas_call(
        _affine_kernel,
        out_shape=jax.ShapeDtypeStruct((n, _C * length), x.dtype),
        grid=(n // tile2,),
        in_specs=[
            smem,
            smem,
            pl.BlockSpec((tile2, _C * length), lambda i: (i, 0)),
        ],
        out_specs=pl.BlockSpec((tile2, _C * length), lambda i: (i, 0)),
        compiler_params=pltpu.CompilerParams(
            dimension_semantics=("parallel",),
            vmem_limit_bytes=_VMEM_BYTES),
    )(scale, shift, y)
    return out2.reshape(n, _C, length)


def kernel(x, w, gamma, beta):
    return _forward(x, w, gamma, beta)


# speedup vs baseline: 1.8428x; 1.0563x over previous
from functools import partial

import jax
import jax.numpy as jnp
from jax.experimental import pallas as pl
from jax.experimental.pallas import tpu as pltpu

_C = 3
_K = 3
_EPS = 1e-5
_VMEM_BYTES = 80 * 1024 * 1024


def _rowtree8(a):
    rows = a.shape[0]
    while rows > 8:
        half = rows // 2
        a = a[:half] + a[half:]
        rows = half
    return a


def _conv_stats_kernel(w_ref, x_ref, y_ref, stats_ref, sacc_ref, qacc_ref):
    step = pl.program_id(1)
    inner = pl.num_programs(1)
    tile_n, _, length = x_ref.shape

    col = jax.lax.broadcasted_iota(jnp.int32, (tile_n, length), 1)
    first = col == 0
    last = col == length - 1

    xs = [x_ref[:, ci, :] for ci in range(_C)]
    taps = []
    for ci in range(_C):
        xc = xs[ci]
        xm = jnp.where(first, 0.0, pltpu.roll(xc, shift=1, axis=1))
        xp = jnp.where(last, 0.0, pltpu.roll(xc, shift=length - 1, axis=1))
        taps.append((xm, xc, xp))

    @pl.when(step == 0)
    def _init():
        sacc_ref[...] = jnp.zeros_like(sacc_ref)
        qacc_ref[...] = jnp.zeros_like(qacc_ref)

    for co in range(_C):
        y = None
        for ci in range(_C):
            base = (co * _C + ci) * _K
            xm, xc, xp = taps[ci]
            t = w_ref[base] * xm + w_ref[base + 1] * xc + w_ref[base + 2] * xp
            y = t if y is None else y + t
        y_ref[co] = y.astype(y_ref.dtype)
        sacc_ref[co] = sacc_ref[co] + _rowtree8(y)
        qacc_ref[co] = qacc_ref[co] + _rowtree8(y * y)

    @pl.when(step == inner - 1)
    def _flush():
        for co in range(_C):
            stats_ref[0, 0, co] = jnp.sum(sacc_ref[co], axis=0)
            stats_ref[0, 1, co] = jnp.sum(qacc_ref[co], axis=0)


def _affine_kernel(sc_ref, sh_ref, y_ref, o_ref):
    for co in range(_C):
        o_ref[:, co, :] = (y_ref[co].astype(jnp.float32) * sc_ref[co]
                           + sh_ref[co])


def _largest_tile(n, cap):
    best = 1
    for t in range(1, min(n, cap) + 1):
        if n % t == 0:
            best = t
    return best


@jax.jit
def _forward(x, w, gamma, beta):
    n, c_in, length = x.shape
    assert c_in == _C and w.shape == (_C, _C, _K)

    tile_n = _largest_tile(n, 256)
    while tile_n > 8 and tile_n & (tile_n - 1):
        tile_n //= 2
    n_tiles = n // tile_n
    n_split = 2 if n_tiles % 2 == 0 else 1
    inner = n_tiles // n_split

    w_flat = w.astype(jnp.float32).reshape(-1)
    smem = pl.BlockSpec(memory_space=pltpu.MemorySpace.SMEM)

    y, stats = pl.pallas_call(
        _conv_stats_kernel,
        out_shape=(
            jax.ShapeDtypeStruct((_C, n, length), jnp.bfloat16),
            jax.ShapeDtypeStruct((n_split, 2, _C, length), jnp.float32),
        ),
        grid=(n_split, inner),
        in_specs=[
            smem,
            pl.BlockSpec((tile_n, _C, length),
                         lambda c, i: (c * inner + i, 0, 0)),
        ],
        out_specs=(
            pl.BlockSpec((_C, tile_n, length),
                         lambda c, i: (0, c * inner + i, 0)),
            pl.BlockSpec((1, 2, _C, length), lambda c, i: (c, 0, 0, 0)),
        ),
        scratch_shapes=[
            pltpu.VMEM((_C, 8, length), jnp.float32),
            pltpu.VMEM((_C, 8, length), jnp.float32),
        ],
        compiler_params=pltpu.CompilerParams(
            dimension_semantics=("parallel", "arbitrary"),
            vmem_limit_bytes=_VMEM_BYTES),
    )(w_flat, x)

    count = jnp.float32(n * length)
    ch_sum = jnp.sum(stats[:, 0], axis=(0, 2))
    ch_sumsq = jnp.sum(stats[:, 1], axis=(0, 2))
    mean = ch_sum / count
    var = jnp.maximum(ch_sumsq / count - mean * mean, 0.0)
    inv = jax.lax.rsqrt(var + _EPS)
    scale = gamma.astype(jnp.float32) * inv
    shift = beta.astype(jnp.float32) - mean * scale

    tile2 = _largest_tile(n, 256)
    out = pl.pallas_call(
        _affine_kernel,
        out_shape=jax.ShapeDtypeStruct((n, _C, length), x.dtype),
        grid=(n // tile2,),
        in_specs=[
            smem,
            smem,
            pl.BlockSpec((_C, tile2, length), lambda i: (0, i, 0)),
        ],
        out_specs=pl.BlockSpec((tile2, _C, length), lambda i: (i, 0, 0)),
        compiler_params=pltpu.CompilerParams(
            dimension_semantics=("parallel",),
            vmem_limit_bytes=_VMEM_BYTES),
    )(scale, shift, y)
    return out


def kernel(x, w, gamma, beta):
    return _forward(x, w, gamma, beta)


# speedup vs baseline: 1.8613x; 1.0100x over previous
import jax
import jax.numpy as jnp
from jax.experimental import pallas as pl
from jax.experimental.pallas import tpu as pltpu

_C = 3
_K = 3
_EPS = 1e-5
_VMEM_BYTES = 80 * 1024 * 1024


def _rowtree8(a):
    rows = a.shape[0]
    while rows > 8:
        half = rows // 2
        a = a[:half] + a[half:]
        rows = half
    return a


def _conv_stats_kernel(w_ref, x_ref, y_ref, stats_ref, sacc_ref, qacc_ref):
    step = pl.program_id(1)
    inner = pl.num_programs(1)
    tile_n, _, length = x_ref.shape

    col = jax.lax.broadcasted_iota(jnp.int32, (tile_n, length), 1)
    first = col == 0
    last = col == length - 1

    xs = [x_ref[:, ci, :] for ci in range(_C)]
    taps = []
    for ci in range(_C):
        xc = xs[ci]
        xm = jnp.where(first, 0.0, pltpu.roll(xc, shift=1, axis=1))
        xp = jnp.where(last, 0.0, pltpu.roll(xc, shift=length - 1, axis=1))
        taps.append((xm, xc, xp))

    @pl.when(step == 0)
    def _init():
        sacc_ref[...] = jnp.zeros_like(sacc_ref)
        qacc_ref[...] = jnp.zeros_like(qacc_ref)

    for co in range(_C):
        y = None
        for ci in range(_C):
            base = (co * _C + ci) * _K
            xm, xc, xp = taps[ci]
            t = w_ref[base] * xm + w_ref[base + 1] * xc + w_ref[base + 2] * xp
            y = t if y is None else y + t
        y_ref[co] = y.astype(y_ref.dtype)
        sacc_ref[co] = sacc_ref[co] + _rowtree8(y)
        qacc_ref[co] = qacc_ref[co] + _rowtree8(y * y)

    @pl.when(step == inner - 1)
    def _flush():
        for co in range(_C):
            stats_ref[0, 0, co] = jnp.sum(sacc_ref[co], axis=0)
            stats_ref[0, 1, co] = jnp.sum(qacc_ref[co], axis=0)


def _affine_kernel(sc_ref, sh_ref, y_ref, o_ref):
    for co in range(_C):
        o_ref[:, co, :] = (y_ref[co].astype(jnp.float32) * sc_ref[co]
                           + sh_ref[co])


def _largest_tile(n, cap):
    best = 1
    for t in range(1, min(n, cap) + 1):
        if n % t == 0:
            best = t
    return best


@jax.jit
def _forward(x, w, gamma, beta):
    n, c_in, length = x.shape
    assert c_in == _C and w.shape == (_C, _C, _K)

    tile_n = _largest_tile(n, 256)
    while tile_n > 8 and tile_n & (tile_n - 1):
        tile_n //= 2
    n_tiles = n // tile_n
    n_split = 2 if n_tiles % 2 == 0 else 1
    inner = n_tiles // n_split

    w_flat = w.astype(jnp.float32).reshape(-1)
    smem = pl.BlockSpec(memory_space=pltpu.MemorySpace.SMEM)

    y, stats = pl.pallas_call(
        _conv_stats_kernel,
        out_shape=(
            jax.ShapeDtypeStruct((_C, n, length), jnp.bfloat16),
            jax.ShapeDtypeStruct((n_split, 2, _C, length), jnp.float32),
        ),
        grid=(n_split, inner),
        in_specs=[
            smem,
            pl.BlockSpec((tile_n, _C, length),
                         lambda c, i: (c * inner + i, 0, 0)),
        ],
        out_specs=(
            pl.BlockSpec((_C, tile_n, length),
                         lambda c, i: (0, c * inner + i, 0)),
            pl.BlockSpec((1, 2, _C, length), lambda c, i: (c, 0, 0, 0)),
        ),
        scratch_shapes=[
            pltpu.VMEM((_C, 8, length), jnp.float32),
            pltpu.VMEM((_C, 8, length), jnp.float32),
        ],
        compiler_params=pltpu.CompilerParams(
            dimension_semantics=("parallel", "arbitrary"),
            vmem_limit_bytes=_VMEM_BYTES),
    )(w_flat, x)

    count = jnp.float32(n * length)
    ch_sum = jnp.sum(stats[:, 0], axis=(0, 2))
    ch_sumsq = jnp.sum(stats[:, 1], axis=(0, 2))
    mean = ch_sum / count
    var = jnp.maximum(ch_sumsq / count - mean * mean, 0.0)
    inv = jax.lax.rsqrt(var + _EPS)
    scale = gamma.astype(jnp.float32) * inv
    shift = beta.astype(jnp.float32) - mean * scale

    tile2 = _largest_tile(n, 512)
    out = pl.pallas_call(
        _affine_kernel,
        out_shape=jax.ShapeDtypeStruct((n, _C, length), x.dtype),
        grid=(n // tile2,),
        in_specs=[
            smem,
            smem,
            pl.BlockSpec((_C, tile2, length), lambda i: (0, i, 0)),
        ],
        out_specs=pl.BlockSpec((tile2, _C, length), lambda i: (i, 0, 0)),
        compiler_params=pltpu.CompilerParams(
            dimension_semantics=("parallel",),
            vmem_limit_bytes=_VMEM_BYTES),
    )(scale, shift, y)
    return out


def kernel(x, w, gamma, beta):
    return _forward(x, w, gamma, beta)


# speedup vs baseline: 1.8674x; 1.0033x over previous
import jax
import jax.numpy as jnp
from jax.experimental import pallas as pl
from jax.experimental.pallas import tpu as pltpu

_C = 3
_K = 3
_EPS = 1e-5
_VMEM_BYTES = 80 * 1024 * 1024


def _rowtree8(a):
    rows = a.shape[0]
    while rows > 8:
        half = rows // 2
        a = a[:half] + a[half:]
        rows = half
    return a


def _conv_stats_kernel(w_ref, x_ref, y_ref, stats_ref, sacc_ref, qacc_ref):
    step = pl.program_id(1)
    inner = pl.num_programs(1)
    tile_n, _, length = x_ref.shape

    col = jax.lax.broadcasted_iota(jnp.int32, (tile_n, length), 1)
    first = col == 0
    last = col == length - 1

    xs = [x_ref[:, ci, :] for ci in range(_C)]
    taps = []
    for ci in range(_C):
        xc = xs[ci]
        xm = jnp.where(first, 0.0, pltpu.roll(xc, shift=1, axis=1))
        xp = jnp.where(last, 0.0, pltpu.roll(xc, shift=length - 1, axis=1))
        taps.append((xm, xc, xp))

    @pl.when(step == 0)
    def _init():
        sacc_ref[...] = jnp.zeros_like(sacc_ref)
        qacc_ref[...] = jnp.zeros_like(qacc_ref)

    for co in range(_C):
        y = None
        for ci in range(_C):
            base = (co * _C + ci) * _K
            xm, xc, xp = taps[ci]
            t = w_ref[base] * xm + w_ref[base + 1] * xc + w_ref[base + 2] * xp
            y = t if y is None else y + t
        y_ref[co] = y.astype(y_ref.dtype)
        sacc_ref[co] = sacc_ref[co] + _rowtree8(y)
        qacc_ref[co] = qacc_ref[co] + _rowtree8(y * y)

    @pl.when(step == inner - 1)
    def _flush():
        for co in range(_C):
            stats_ref[0, 0, co] = jnp.sum(sacc_ref[co], axis=0)
            stats_ref[0, 1, co] = jnp.sum(qacc_ref[co], axis=0)


def _affine_kernel(sc_ref, sh_ref, y_ref, o_ref):
    for co in range(_C):
        o_ref[:, co, :] = (y_ref[co].astype(jnp.float32) * sc_ref[co]
                           + sh_ref[co])


def _largest_tile(n, cap):
    best = 1
    for t in range(1, min(n, cap) + 1):
        if n % t == 0:
            best = t
    return best


@jax.jit
def _forward(x, w, gamma, beta):
    n, c_in, length = x.shape
    assert c_in == _C and w.shape == (_C, _C, _K)

    tile_n = _largest_tile(n, 256)
    while tile_n > 8 and tile_n & (tile_n - 1):
        tile_n //= 2
    n_tiles = n // tile_n
    n_split = 2 if n_tiles % 2 == 0 else 1
    inner = n_tiles // n_split

    w_flat = w.astype(jnp.float32).reshape(-1)
    smem = pl.BlockSpec(memory_space=pltpu.MemorySpace.SMEM)

    y, stats = pl.pallas_call(
        _conv_stats_kernel,
        out_shape=(
            jax.ShapeDtypeStruct((_C, n, length), jnp.bfloat16),
            jax.ShapeDtypeStruct((n_split, 2, _C, length), jnp.float32),
        ),
        grid=(n_split, inner),
        in_specs=[
            smem,
            pl.BlockSpec((tile_n, _C, length),
                         lambda c, i: (c * inner + i, 0, 0)),
        ],
        out_specs=(
            pl.BlockSpec((_C, tile_n, length),
                         lambda c, i: (0, c * inner + i, 0)),
            pl.BlockSpec((1, 2, _C, length), lambda c, i: (c, 0, 0, 0)),
        ),
        scratch_shapes=[
            pltpu.VMEM((_C, 8, length), jnp.float32),
            pltpu.VMEM((_C, 8, length), jnp.float32),
        ],
        compiler_params=pltpu.CompilerParams(
            dimension_semantics=("parallel", "arbitrary"),
            vmem_limit_bytes=_VMEM_BYTES),
    )(w_flat, x)

    count = jnp.float32(n * length)
    ch_sum = jnp.sum(stats[:, 0], axis=(0, 2))
    ch_sumsq = jnp.sum(stats[:, 1], axis=(0, 2))
    mean = ch_sum / count
    var = jnp.maximum(ch_sumsq / count - mean * mean, 0.0)
    inv = jax.lax.rsqrt(var + _EPS)
    scale = gamma.astype(jnp.float32) * inv
    shift = beta.astype(jnp.float32) - mean * scale

    tile2 = _largest_tile(n, 1024)
    out = pl.pallas_call(
        _affine_kernel,
        out_shape=jax.ShapeDtypeStruct((n, _C, length), x.dtype),
        grid=(n // tile2,),
        in_specs=[
            smem,
            smem,
            pl.BlockSpec((_C, tile2, length), lambda i: (0, i, 0)),
        ],
        out_specs=pl.BlockSpec((tile2, _C, length), lambda i: (i, 0, 0)),
        compiler_params=pltpu.CompilerParams(
            dimension_semantics=("parallel",),
            vmem_limit_bytes=_VMEM_BYTES),
    )(scale, shift, y)
    return out


def kernel(x, w, gamma, beta):
    return _forward(x, w, gamma, beta)
